# speedup vs baseline: 16.7468x; 16.7468x over previous
import functools

import jax
import jax.numpy as jnp
from jax import lax
from jax.experimental import pallas as pl
from jax.experimental.pallas import tpu as pltpu
from jax.experimental.pallas import tpu_sc as plsc

_NSC = 2
_NSUB = 16
_NW = _NSC * _NSUB
_CH = 128
_BR = 1024


def _sc_mesh():
    return plsc.VectorSubcoreMesh(core_axis_name="c", subcore_axis_name="s")


def _fill_zeros_1d(ref, cols):
    z16 = jnp.zeros((16,), jnp.float32)
    for k in range(cols // 16):
        ref[pl.ds(k * 16, 16)] = z16


def _fill_zeros_2d(ref, rows, cols):
    z16 = jnp.zeros((16,), jnp.float32)
    for r in range(rows):
        for k in range(cols // 16):
            ref[r, pl.ds(k * 16, 16)] = z16


@functools.partial(jax.jit, static_argnames=("n_pad", "e_pad"))
def _sc_degree(dst, *, n_pad, e_pad):
    rows_per_sub = n_pad // _NSUB
    e_per_w = e_pad // _NW
    nch = e_per_w // _CH

    @functools.partial(
        pl.kernel,
        mesh=_sc_mesh(),
        out_type=jax.ShapeDtypeStruct((_NSC, n_pad), jnp.float32),
        scratch_types=[
            pltpu.VMEM((_CH,), jnp.int32),
            pltpu.VMEM((_CH,), jnp.float32),
            pltpu.VMEM((rows_per_sub,), jnp.float32),
            pltpu.VMEM_SHARED((n_pad,), jnp.float32),
        ],
    )
    def deg_kernel(dst_hbm, out_hbm, dst_v, ones_v, zb, deg_sh):
        c = lax.axis_index("c")
        s = lax.axis_index("s")
        wid = c * _NSUB + s
        one16 = jnp.ones((16,), jnp.float32)
        for k in range(_CH // 16):
            ones_v[pl.ds(k * 16, 16)] = one16
        _fill_zeros_1d(zb, rows_per_sub)
        r0 = s * rows_per_sub
        pltpu.sync_copy(zb, deg_sh.at[pl.ds(r0, rows_per_sub)])
        plsc.subcore_barrier()

        base = wid * e_per_w

        def body(i, carry):
            off = base + i * _CH
            pltpu.sync_copy(dst_hbm.at[pl.ds(off, _CH)], dst_v)
            pltpu.sync_copy(ones_v, deg_sh.at[dst_v], add=True)
            return carry

        lax.fori_loop(0, nch, body, 0)
        plsc.subcore_barrier()
        pltpu.sync_copy(deg_sh.at[pl.ds(r0, rows_per_sub)],
                        out_hbm.at[c, pl.ds(r0, rows_per_sub)])

    return deg_kernel(dst)


@functools.partial(jax.jit, static_argnames=("n_pad", "e_pad", "d"))
def _sc_aggregate(hs, src, dst, *, n_pad, e_pad, d):
    rows_per_sub = n_pad // _NSUB
    e_per_w = e_pad // _NW
    nch = e_per_w // _CH

    @functools.partial(
        pl.kernel,
        mesh=_sc_mesh(),
        out_type=jax.ShapeDtypeStruct((_NSC, n_pad, d), jnp.float32),
        scratch_types=[
            pltpu.VMEM((_CH,), jnp.int32),
            pltpu.VMEM((_CH,), jnp.int32),
            pltpu.VMEM((_CH, d), jnp.float32),
            pltpu.VMEM((16, d), jnp.float32),
            pltpu.VMEM_SHARED((n_pad, d), jnp.float32),
            pltpu.SemaphoreType.DMA,
        ],
    )
    def agg_kernel(hs_hbm, src_hbm, dst_hbm, out_hbm,
                   src_v, dst_v, data_v, zbuf, acc_sh, sem):
        c = lax.axis_index("c")
        s = lax.axis_index("s")
        wid = c * _NSUB + s
        _fill_zeros_2d(zbuf, 16, d)
        r0 = s * rows_per_sub

        def zloop(i, carry):
            pltpu.sync_copy(zbuf, acc_sh.at[pl.ds(r0 + i * 16, 16)])
            return carry

        lax.fori_loop(0, rows_per_sub // 16, zloop, 0)
        plsc.subcore_barrier()

        base = wid * e_per_w

        def body(i, carry):
            off = base + i * _CH
            pltpu.sync_copy(src_hbm.at[pl.ds(off, _CH)], src_v)
            pltpu.sync_copy(dst_hbm.at[pl.ds(off, _CH)], dst_v)
            pltpu.async_copy(hs_hbm.at[src_v], data_v, sem).wait()
            pltpu.sync_copy(data_v, acc_sh.at[dst_v], add=True)
            return carry

        lax.fori_loop(0, nch, body, 0)
        plsc.subcore_barrier()
        pltpu.sync_copy(acc_sh.at[pl.ds(r0, rows_per_sub)],
                        out_hbm.at[c, pl.ds(r0, rows_per_sub)])

    return agg_kernel(hs, src, dst)


def _tc_prep(x_pad, W1, degp, *, n_pad, f_in, h):
    def body(x_ref, w_ref, degp_ref, hs_ref, dinv_ref):
        deg = degp_ref[0, :] + degp_ref[1, :]
        dinv = lax.rsqrt(deg + 1.0)
        hw = jnp.dot(x_ref[...], w_ref[...], preferred_element_type=jnp.float32)
        hs_ref[...] = hw * dinv[:, None]
        dinv_ref[...] = dinv

    return pl.pallas_call(
        body,
        grid=(n_pad // _BR,),
        in_specs=[
            pl.BlockSpec((_BR, f_in), lambda i: (i, 0)),
            pl.BlockSpec((f_in, h), lambda i: (0, 0)),
            pl.BlockSpec((_NSC, _BR), lambda i: (0, i)),
        ],
        out_specs=[
            pl.BlockSpec((_BR, h), lambda i: (i, 0)),
            pl.BlockSpec((_BR,), lambda i: (i,)),
        ],
        out_shape=[
            jax.ShapeDtypeStruct((n_pad, h), jnp.float32),
            jax.ShapeDtypeStruct((n_pad,), jnp.float32),
        ],
    )(x_pad, W1, degp)


def _tc_mid(p1, hs1, dinv, b1, W2, *, n_pad, h):
    def body(p_ref, hs1_ref, dinv_ref, b1_ref, w2_ref, hs2_ref):
        dinv = dinv_ref[...]
        agg = p_ref[0] + p_ref[1] + hs1_ref[...]
        h1 = jnp.maximum(agg * dinv[:, None] + b1_ref[...][None, :], 0.0)
        hs2_ref[...] = (
            jnp.dot(h1, w2_ref[...], preferred_element_type=jnp.float32)
            * dinv[:, None])

    return pl.pallas_call(
        body,
        grid=(n_pad // _BR,),
        in_specs=[
            pl.BlockSpec((_NSC, _BR, h), lambda i: (0, i, 0)),
            pl.BlockSpec((_BR, h), lambda i: (i, 0)),
            pl.BlockSpec((_BR,), lambda i: (i,)),
            pl.BlockSpec((h,), lambda i: (0,)),
            pl.BlockSpec((h, h), lambda i: (0, 0)),
        ],
        out_specs=pl.BlockSpec((_BR, h), lambda i: (i, 0)),
        out_shape=jax.ShapeDtypeStruct((n_pad, h), jnp.float32),
    )(p1, hs1, dinv, b1, W2)


def _tc_final(p2, hs2, dinv, b2, batch_pad, gf, Wl1a, Wl1b, bl1, Wl2, bl2,
              *, n_pad, h, ng, g, nc):
    nblocks = n_pad // _BR

    def body(p_ref, hs2_ref, dinv_ref, b2_ref, batch_ref, gf_ref,
             wl1a_ref, wl1b_ref, bl1_ref, wl2_ref, bl2_ref,
             z_ref, pooled_acc, counts_acc):
        i = pl.program_id(0)

        @pl.when(i == 0)
        def _():
            pooled_acc[...] = jnp.zeros_like(pooled_acc)
            counts_acc[...] = jnp.zeros_like(counts_acc)

        dinv = dinv_ref[...]
        h2 = ((p_ref[0] + p_ref[1] + hs2_ref[...]) * dinv[:, None]
              + b2_ref[...][None, :])
        b = batch_ref[...]
        gids = lax.broadcasted_iota(jnp.int32, (ng, _BR), 0)
        onehot = (b[None, :] == gids).astype(jnp.float32)
        pooled_acc[...] += jnp.dot(onehot, h2,
                                   preferred_element_type=jnp.float32)
        counts_acc[...] += jnp.sum(onehot, axis=1)[:, None]

        @pl.when(i == nblocks - 1)
        def _():
            pooled = pooled_acc[...] / jnp.maximum(counts_acc[...], 1.0)
            t = (jnp.dot(pooled, wl1a_ref[...],
                         preferred_element_type=jnp.float32)
                 + jnp.dot(gf_ref[...], wl1b_ref[...],
                           preferred_element_type=jnp.float32)
                 + bl1_ref[...][None, :])
            t = jnp.maximum(t, 0.0)
            z_ref[...] = (jnp.dot(t, wl2_ref[...],
                                  preferred_element_type=jnp.float32)
                          + bl2_ref[...][None, :])

    return pl.pallas_call(
        body,
        grid=(nblocks,),
        in_specs=[
            pl.BlockSpec((_NSC, _BR, h), lambda i: (0, i, 0)),
            pl.BlockSpec((_BR, h), lambda i: (i, 0)),
            pl.BlockSpec((_BR,), lambda i: (i,)),
            pl.BlockSpec((h,), lambda i: (0,)),
            pl.BlockSpec((_BR,), lambda i: (i,)),
            pl.BlockSpec((ng, g), lambda i: (0, 0)),
            pl.BlockSpec((h, h), lambda i: (0, 0)),
            pl.BlockSpec((g, h), lambda i: (0, 0)),
            pl.BlockSpec((h,), lambda i: (0,)),
            pl.BlockSpec((h, nc), lambda i: (0, 0)),
            pl.BlockSpec((nc,), lambda i: (0,)),
        ],
        out_specs=pl.BlockSpec((ng, nc), lambda i: (0, 0)),
        out_shape=jax.ShapeDtypeStruct((ng, nc), jnp.float32),
        scratch_shapes=[
            pltpu.VMEM((ng, h), jnp.float32),
            pltpu.VMEM((ng, h), jnp.float32),
        ],
    )(p2, hs2, dinv, b2, batch_pad, gf, Wl1a, Wl1b, bl1, Wl2, bl2)


def kernel(x, edge_index, batch, global_features, W1, b1, W2, b2,
           Wl1, bl1, Wl2, bl2):
    n, f_in = x.shape
    h = W1.shape[1]
    e = edge_index.shape[1]
    ng, g = global_features.shape
    nc = Wl2.shape[1]

    n_pad = -(-(n + 1) // _BR) * _BR
    e_pad = -(-e // (_NW * _CH)) * (_NW * _CH)

    n_dump = n_pad - n
    pad_idx = n + jnp.arange(e_pad - e, dtype=jnp.int32) % n_dump
    src = jnp.concatenate([edge_index[0], pad_idx])
    dst = jnp.concatenate([edge_index[1], pad_idx])

    x_pad = jnp.pad(x, ((0, n_pad - n), (0, 0)))
    batch_pad = jnp.pad(batch, (0, n_pad - n), constant_values=-1)

    degp = _sc_degree(dst, n_pad=n_pad, e_pad=e_pad)
    hs1, dinv = _tc_prep(x_pad, W1, degp, n_pad=n_pad, f_in=f_in, h=h)
    p1 = _sc_aggregate(hs1, src, dst, n_pad=n_pad, e_pad=e_pad, d=h)
    hs2 = _tc_mid(p1, hs1, dinv, b1, W2, n_pad=n_pad, h=h)
    p2 = _sc_aggregate(hs2, src, dst, n_pad=n_pad, e_pad=e_pad, d=h)
    z = _tc_final(p2, hs2, dinv, b2, batch_pad, global_features,
                  Wl1[:h], Wl1[h:], bl1, Wl2, bl2,
                  n_pad=n_pad, h=h, ng=ng, g=g, nc=nc)
    return z


# speedup vs baseline: 30.0997x; 1.7973x over previous
import functools

import jax
import jax.numpy as jnp
from jax import lax
from jax.experimental import pallas as pl
from jax.experimental.pallas import tpu as pltpu
from jax.experimental.pallas import tpu_sc as plsc

_NSC = 2
_NSUB = 16
_NW = _NSC * _NSUB
_CH = 128
_BR = 1024


def _sc_mesh():
    return plsc.VectorSubcoreMesh(core_axis_name="c", subcore_axis_name="s")


def _fill_zeros_1d(ref, cols):
    z16 = jnp.zeros((16,), jnp.float32)
    for k in range(cols // 16):
        ref[pl.ds(k * 16, 16)] = z16


def _fill_zeros_2d(ref, rows, cols):
    z16 = jnp.zeros((16,), jnp.float32)
    for r in range(rows):
        for k in range(cols // 16):
            ref[r, pl.ds(k * 16, 16)] = z16


@functools.partial(jax.jit, static_argnames=("n_pad", "e_pad"))
def _sc_degree(dst, *, n_pad, e_pad):
    rows_per_sub = n_pad // _NSUB
    e_per_w = e_pad // _NW
    nch = e_per_w // _CH

    @functools.partial(
        pl.kernel,
        mesh=_sc_mesh(),
        out_type=jax.ShapeDtypeStruct((_NSC, n_pad), jnp.float32),
        scratch_types=[
            pltpu.VMEM((_CH,), jnp.int32),
            pltpu.VMEM((_CH,), jnp.float32),
            pltpu.VMEM((rows_per_sub,), jnp.float32),
            pltpu.VMEM_SHARED((n_pad,), jnp.float32),
        ],
    )
    def deg_kernel(dst_hbm, out_hbm, dst_v, ones_v, zb, deg_sh):
        c = lax.axis_index("c")
        s = lax.axis_index("s")
        wid = c * _NSUB + s
        one16 = jnp.ones((16,), jnp.float32)
        for k in range(_CH // 16):
            ones_v[pl.ds(k * 16, 16)] = one16
        _fill_zeros_1d(zb, rows_per_sub)
        r0 = s * rows_per_sub
        pltpu.sync_copy(zb, deg_sh.at[pl.ds(r0, rows_per_sub)])
        plsc.subcore_barrier()

        base = wid * e_per_w

        def body(i, carry):
            off = base + i * _CH
            pltpu.sync_copy(dst_hbm.at[pl.ds(off, _CH)], dst_v)
            pltpu.sync_copy(ones_v, deg_sh.at[dst_v], add=True)
            return carry

        lax.fori_loop(0, nch, body, 0)
        plsc.subcore_barrier()
        pltpu.sync_copy(deg_sh.at[pl.ds(r0, rows_per_sub)],
                        out_hbm.at[c, pl.ds(r0, rows_per_sub)])

    return deg_kernel(dst)


_SG = 8


@functools.partial(jax.jit, static_argnames=("n_pad", "e_pad", "d"))
def _sc_aggregate(hs, src2, dst2, *, n_pad, e_pad, d):
    rows_per_sub = n_pad // _NSUB
    e_per_w = e_pad // _NW
    nch = e_per_w // _CH
    nsg = nch // _SG
    assert nch == nsg * _SG and nsg >= 2
    total_rows = e_pad // _CH

    @functools.partial(
        pl.kernel,
        mesh=_sc_mesh(),
        out_type=jax.ShapeDtypeStruct((_NSC, n_pad, d), jnp.float32),
        scratch_types=[
            pltpu.VMEM((2 * _SG, _CH), jnp.int32),
            pltpu.VMEM((2 * _SG, _CH), jnp.int32),
            pltpu.VMEM((_CH, d), jnp.float32),
            pltpu.VMEM((_CH, d), jnp.float32),
            pltpu.VMEM((8, d), jnp.float32),
            pltpu.VMEM_SHARED((n_pad, d), jnp.float32),
            pltpu.SemaphoreType.DMA,
            pltpu.SemaphoreType.DMA,
            pltpu.SemaphoreType.DMA,
            pltpu.SemaphoreType.DMA,
            pltpu.SemaphoreType.DMA,
        ],
    )
    def agg_kernel(hs_hbm, src_hbm, dst_hbm, out_hbm,
                   src_i, dst_i, d0, d1, zbuf, acc_sh,
                   sg0, sg1, ss0, ss1, si):
        data = (d0, d1)
        gsem = (sg0, sg1)
        ssem = (ss0, ss1)
        c = lax.axis_index("c")
        s = lax.axis_index("s")
        wid = c * _NSUB + s
        row0 = wid * nch

        def g_start(row, b):
            pltpu.async_copy(hs_hbm.at[src_i.at[row]], data[b], gsem[b])

        def g_wait(b):
            pltpu.make_async_copy(hs_hbm.at[pl.ds(0, _CH)], data[b],
                                  gsem[b]).wait()

        def s_start(row, b):
            pltpu.async_copy(data[b], acc_sh.at[dst_i.at[row]], ssem[b],
                             add=True)

        def s_wait(b):
            pltpu.make_async_copy(data[b], acc_sh.at[pl.ds(0, _CH)],
                                  ssem[b]).wait()

        def idx_prefetch(sg_next, half_base):
            rstart = jnp.minimum(row0 + sg_next * _SG,
                                 jnp.int32(total_rows - _SG))
            pltpu.async_copy(src_hbm.at[pl.ds(rstart, _SG)],
                             src_i.at[pl.ds(half_base, _SG)], si)
            pltpu.async_copy(dst_hbm.at[pl.ds(rstart, _SG)],
                             dst_i.at[pl.ds(half_base, _SG)], si)

        def idx_wait():
            pltpu.make_async_copy(src_hbm.at[pl.ds(0, _SG)],
                                  src_i.at[pl.ds(0, _SG)], si).wait()
            pltpu.make_async_copy(dst_hbm.at[pl.ds(0, _SG)],
                                  dst_i.at[pl.ds(0, _SG)], si).wait()

        pltpu.sync_copy(src_hbm.at[pl.ds(row0, _SG)],
                        src_i.at[pl.ds(0, _SG)])
        pltpu.sync_copy(dst_hbm.at[pl.ds(row0, _SG)],
                        dst_i.at[pl.ds(0, _SG)])
        _fill_zeros_2d(zbuf, 8, d)
        r0 = s * rows_per_sub

        def zloop(i, carry):
            pltpu.sync_copy(zbuf, acc_sh.at[pl.ds(r0 + i * 8, 8)])
            return carry

        lax.fori_loop(0, rows_per_sub // 8, zloop, 0)
        plsc.subcore_barrier()

        for k in range(_SG):
            b = k % 2
            if k >= 2:
                s_wait(b)
            g_start(k, b)
            if k >= 1:
                g_wait(b ^ 1)
                s_start(k - 1, b ^ 1)
        idx_prefetch(1, _SG)

        def body(sg, carry):
            idx_wait()
            hb = lax.rem(sg, 2) * _SG
            ob = _SG - hb
            for k in range(_SG):
                b = k % 2
                s_wait(b)
                g_start(hb + k, b)
                g_wait(b ^ 1)
                if k == 0:
                    s_start(ob + _SG - 1, b ^ 1)
                else:
                    s_start(hb + k - 1, b ^ 1)
            idx_prefetch(sg + 1, ob)
            return carry

        lax.fori_loop(1, nsg, body, 0)

        hb_last = ((nsg - 1) % 2) * _SG
        g_wait(1)
        s_start(hb_last + _SG - 1, 1)
        s_wait(0)
        s_wait(1)
        idx_wait()

        plsc.subcore_barrier()
        pltpu.sync_copy(acc_sh.at[pl.ds(r0, rows_per_sub)],
                        out_hbm.at[c, pl.ds(r0, rows_per_sub)])

    return agg_kernel(hs, src2, dst2)


def _tc_prep(x_pad, W1, degp, *, n_pad, f_in, h):
    def body(x_ref, w_ref, degp_ref, hs_ref, dinv_ref):
        deg = degp_ref[0, :] + degp_ref[1, :]
        dinv = lax.rsqrt(deg + 1.0)
        hw = jnp.dot(x_ref[...], w_ref[...], preferred_element_type=jnp.float32)
        hs_ref[...] = hw * dinv[:, None]
        dinv_ref[...] = dinv

    return pl.pallas_call(
        body,
        grid=(n_pad // _BR,),
        in_specs=[
            pl.BlockSpec((_BR, f_in), lambda i: (i, 0)),
            pl.BlockSpec((f_in, h), lambda i: (0, 0)),
            pl.BlockSpec((_NSC, _BR), lambda i: (0, i)),
        ],
        out_specs=[
            pl.BlockSpec((_BR, h), lambda i: (i, 0)),
            pl.BlockSpec((_BR,), lambda i: (i,)),
        ],
        out_shape=[
            jax.ShapeDtypeStruct((n_pad, h), jnp.float32),
            jax.ShapeDtypeStruct((n_pad,), jnp.float32),
        ],
    )(x_pad, W1, degp)


def _tc_mid(p1, hs1, dinv, b1, W2, *, n_pad, h):
    def body(p_ref, hs1_ref, dinv_ref, b1_ref, w2_ref, hs2_ref):
        dinv = dinv_ref[...]
        agg = p_ref[0] + p_ref[1] + hs1_ref[...]
        h1 = jnp.maximum(agg * dinv[:, None] + b1_ref[...][None, :], 0.0)
        hs2_ref[...] = (
            jnp.dot(h1, w2_ref[...], preferred_element_type=jnp.float32)
            * dinv[:, None])

    return pl.pallas_call(
        body,
        grid=(n_pad // _BR,),
        in_specs=[
            pl.BlockSpec((_NSC, _BR, h), lambda i: (0, i, 0)),
            pl.BlockSpec((_BR, h), lambda i: (i, 0)),
            pl.BlockSpec((_BR,), lambda i: (i,)),
            pl.BlockSpec((h,), lambda i: (0,)),
            pl.BlockSpec((h, h), lambda i: (0, 0)),
        ],
        out_specs=pl.BlockSpec((_BR, h), lambda i: (i, 0)),
        out_shape=jax.ShapeDtypeStruct((n_pad, h), jnp.float32),
    )(p1, hs1, dinv, b1, W2)


def _tc_final(p2, hs2, dinv, b2, batch_pad, gf, Wl1a, Wl1b, bl1, Wl2, bl2,
              *, n_pad, h, ng, g, nc):
    nblocks = n_pad // _BR

    def body(p_ref, hs2_ref, dinv_ref, b2_ref, batch_ref, gf_ref,
             wl1a_ref, wl1b_ref, bl1_ref, wl2_ref, bl2_ref,
             z_ref, pooled_acc, counts_acc):
        i = pl.program_id(0)

        @pl.when(i == 0)
        def _():
            pooled_acc[...] = jnp.zeros_like(pooled_acc)
            counts_acc[...] = jnp.zeros_like(counts_acc)

        dinv = dinv_ref[...]
        h2 = ((p_ref[0] + p_ref[1] + hs2_ref[...]) * dinv[:, None]
              + b2_ref[...][None, :])
        b = batch_ref[...]
        gids = lax.broadcasted_iota(jnp.int32, (ng, _BR), 0)
        onehot = (b[None, :] == gids).astype(jnp.float32)
        pooled_acc[...] += jnp.dot(onehot, h2,
                                   preferred_element_type=jnp.float32)
        counts_acc[...] += jnp.sum(onehot, axis=1)[:, None]

        @pl.when(i == nblocks - 1)
        def _():
            pooled = pooled_acc[...] / jnp.maximum(counts_acc[...], 1.0)
            t = (jnp.dot(pooled, wl1a_ref[...],
                         preferred_element_type=jnp.float32)
                 + jnp.dot(gf_ref[...], wl1b_ref[...],
                           preferred_element_type=jnp.float32)
                 + bl1_ref[...][None, :])
            t = jnp.maximum(t, 0.0)
            z_ref[...] = (jnp.dot(t, wl2_ref[...],
                                  preferred_element_type=jnp.float32)
                          + bl2_ref[...][None, :])

    return pl.pallas_call(
        body,
        grid=(nblocks,),
        in_specs=[
            pl.BlockSpec((_NSC, _BR, h), lambda i: (0, i, 0)),
            pl.BlockSpec((_BR, h), lambda i: (i, 0)),
            pl.BlockSpec((_BR,), lambda i: (i,)),
            pl.BlockSpec((h,), lambda i: (0,)),
            pl.BlockSpec((_BR,), lambda i: (i,)),
            pl.BlockSpec((ng, g), lambda i: (0, 0)),
            pl.BlockSpec((h, h), lambda i: (0, 0)),
            pl.BlockSpec((g, h), lambda i: (0, 0)),
            pl.BlockSpec((h,), lambda i: (0,)),
            pl.BlockSpec((h, nc), lambda i: (0, 0)),
            pl.BlockSpec((nc,), lambda i: (0,)),
        ],
        out_specs=pl.BlockSpec((ng, nc), lambda i: (0, 0)),
        out_shape=jax.ShapeDtypeStruct((ng, nc), jnp.float32),
        scratch_shapes=[
            pltpu.VMEM((ng, h), jnp.float32),
            pltpu.VMEM((ng, h), jnp.float32),
        ],
    )(p2, hs2, dinv, b2, batch_pad, gf, Wl1a, Wl1b, bl1, Wl2, bl2)


def kernel(x, edge_index, batch, global_features, W1, b1, W2, b2,
           Wl1, bl1, Wl2, bl2):
    n, f_in = x.shape
    h = W1.shape[1]
    e = edge_index.shape[1]
    ng, g = global_features.shape
    nc = Wl2.shape[1]

    n_pad = -(-(n + 1) // _BR) * _BR
    e_chunk = _NW * _CH * _SG
    e_pad = -(-e // e_chunk) * e_chunk

    n_dump = n_pad - n
    pad_idx = n + jnp.arange(e_pad - e, dtype=jnp.int32) % n_dump
    src = jnp.concatenate([edge_index[0], pad_idx])
    dst = jnp.concatenate([edge_index[1], pad_idx])

    x_pad = jnp.pad(x, ((0, n_pad - n), (0, 0)))
    batch_pad = jnp.pad(batch, (0, n_pad - n), constant_values=-1)

    src2 = src.reshape(e_pad // _CH, _CH)
    dst2 = dst.reshape(e_pad // _CH, _CH)

    degp = _sc_degree(dst, n_pad=n_pad, e_pad=e_pad)
    hs1, dinv = _tc_prep(x_pad, W1, degp, n_pad=n_pad, f_in=f_in, h=h)
    p1 = _sc_aggregate(hs1, src2, dst2, n_pad=n_pad, e_pad=e_pad, d=h)
    hs2 = _tc_mid(p1, hs1, dinv, b1, W2, n_pad=n_pad, h=h)
    p2 = _sc_aggregate(hs2, src2, dst2, n_pad=n_pad, e_pad=e_pad, d=h)
    z = _tc_final(p2, hs2, dinv, b2, batch_pad, global_features,
                  Wl1[:h], Wl1[h:], bl1, Wl2, bl2,
                  n_pad=n_pad, h=h, ng=ng, g=g, nc=nc)
    return z


# speedup vs baseline: 33.8856x; 1.1258x over previous
import functools

import jax
import jax.numpy as jnp
from jax import lax
from jax.experimental import pallas as pl
from jax.experimental.pallas import tpu as pltpu
from jax.experimental.pallas import tpu_sc as plsc

_NSC = 2
_NSUB = 16
_NW = _NSC * _NSUB
_CH = 128
_BR = 1024


def _sc_mesh():
    return plsc.VectorSubcoreMesh(core_axis_name="c", subcore_axis_name="s")


def _fill_zeros_1d(ref, cols):
    z16 = jnp.zeros((16,), jnp.float32)
    for k in range(cols // 16):
        ref[pl.ds(k * 16, 16)] = z16


def _fill_zeros_2d(ref, rows, cols):
    z16 = jnp.zeros((16,), jnp.float32)
    for r in range(rows):
        for k in range(cols // 16):
            ref[r, pl.ds(k * 16, 16)] = z16


@functools.partial(jax.jit, static_argnames=("n_pad", "e_pad"))
def _sc_degree(dst2, *, n_pad, e_pad):
    rows_per_sub = n_pad // _NSUB
    e_per_w = e_pad // _NW
    nch = e_per_w // _CH
    nsg = nch // _SG
    assert nch == nsg * _SG and nsg >= 2
    total_rows = e_pad // _CH

    @functools.partial(
        pl.kernel,
        mesh=_sc_mesh(),
        out_type=jax.ShapeDtypeStruct((_NSC, n_pad), jnp.float32),
        scratch_types=[
            pltpu.VMEM((2 * _SG, _CH), jnp.int32),
            pltpu.VMEM((_CH,), jnp.float32),
            pltpu.VMEM((rows_per_sub,), jnp.float32),
            pltpu.VMEM_SHARED((n_pad,), jnp.float32),
            pltpu.SemaphoreType.DMA,
            pltpu.SemaphoreType.DMA,
        ],
    )
    def deg_kernel(dst_hbm, out_hbm, dst_i, ones_v, zb, deg_sh, ss, si):
        c = lax.axis_index("c")
        s = lax.axis_index("s")
        wid = c * _NSUB + s
        row0 = wid * nch
        one16 = jnp.ones((16,), jnp.float32)
        for k in range(_CH // 16):
            ones_v[pl.ds(k * 16, 16)] = one16

        def s_start(row):
            pltpu.async_copy(ones_v, deg_sh.at[dst_i.at[row]], ss, add=True)

        def s_drain():
            for _ in range(_SG):
                pltpu.make_async_copy(ones_v, deg_sh.at[pl.ds(0, _CH)],
                                      ss).wait()

        def idx_prefetch(sg_next, half_base):
            rstart = jnp.minimum(row0 + sg_next * _SG,
                                 jnp.int32(total_rows - _SG))
            pltpu.async_copy(dst_hbm.at[pl.ds(rstart, _SG)],
                             dst_i.at[pl.ds(half_base, _SG)], si)

        def idx_wait():
            pltpu.make_async_copy(dst_hbm.at[pl.ds(0, _SG)],
                                  dst_i.at[pl.ds(0, _SG)], si).wait()

        pltpu.sync_copy(dst_hbm.at[pl.ds(row0, _SG)],
                        dst_i.at[pl.ds(0, _SG)])
        _fill_zeros_1d(zb, rows_per_sub)
        r0 = s * rows_per_sub
        pltpu.sync_copy(zb, deg_sh.at[pl.ds(r0, rows_per_sub)])
        plsc.subcore_barrier()

        for k in range(_SG):
            s_start(k)
        idx_prefetch(1, _SG)

        def body(sg, carry):
            idx_wait()
            hb = lax.rem(sg, 2) * _SG
            s_drain()
            for k in range(_SG):
                s_start(hb + k)
            idx_prefetch(sg + 1, _SG - hb)
            return carry

        lax.fori_loop(1, nsg, body, 0)
        s_drain()
        idx_wait()

        plsc.subcore_barrier()
        pltpu.sync_copy(deg_sh.at[pl.ds(r0, rows_per_sub)],
                        out_hbm.at[c, pl.ds(r0, rows_per_sub)])

    return deg_kernel(dst2)


_SG = 8


@functools.partial(jax.jit, static_argnames=("n_pad", "e_pad", "d"))
def _sc_aggregate(hs, src2, dst2, *, n_pad, e_pad, d):
    rows_per_sub = n_pad // _NSUB
    e_per_w = e_pad // _NW
    nch = e_per_w // _CH
    nsg = nch // _SG
    assert nch == nsg * _SG and nsg >= 2
    total_rows = e_pad // _CH

    @functools.partial(
        pl.kernel,
        mesh=_sc_mesh(),
        out_type=jax.ShapeDtypeStruct((_NSC, n_pad, d), jnp.float32),
        scratch_types=[
            pltpu.VMEM((2 * _SG, _CH), jnp.int32),
            pltpu.VMEM((2 * _SG, _CH), jnp.int32),
            pltpu.VMEM((_CH, d), jnp.float32),
            pltpu.VMEM((_CH, d), jnp.float32),
            pltpu.VMEM((8, d), jnp.float32),
            pltpu.VMEM_SHARED((n_pad, d), jnp.float32),
            pltpu.SemaphoreType.DMA,
            pltpu.SemaphoreType.DMA,
            pltpu.SemaphoreType.DMA,
            pltpu.SemaphoreType.DMA,
            pltpu.SemaphoreType.DMA,
        ],
    )
    def agg_kernel(hs_hbm, src_hbm, dst_hbm, out_hbm,
                   src_i, dst_i, d0, d1, zbuf, acc_sh,
                   sg0, sg1, ss0, ss1, si):
        data = (d0, d1)
        gsem = (sg0, sg1)
        ssem = (ss0, ss1)
        c = lax.axis_index("c")
        s = lax.axis_index("s")
        wid = c * _NSUB + s
        row0 = wid * nch

        def g_start(row, b):
            pltpu.async_copy(hs_hbm.at[src_i.at[row]], data[b], gsem[b])

        def g_wait(b):
            pltpu.make_async_copy(hs_hbm.at[pl.ds(0, _CH)], data[b],
                                  gsem[b]).wait()

        def s_start(row, b):
            pltpu.async_copy(data[b], acc_sh.at[dst_i.at[row]], ssem[b],
                             add=True)

        def s_wait(b):
            pltpu.make_async_copy(data[b], acc_sh.at[pl.ds(0, _CH)],
                                  ssem[b]).wait()

        def idx_prefetch(sg_next, half_base):
            rstart = jnp.minimum(row0 + sg_next * _SG,
                                 jnp.int32(total_rows - _SG))
            pltpu.async_copy(src_hbm.at[pl.ds(rstart, _SG)],
                             src_i.at[pl.ds(half_base, _SG)], si)
            pltpu.async_copy(dst_hbm.at[pl.ds(rstart, _SG)],
                             dst_i.at[pl.ds(half_base, _SG)], si)

        def idx_wait():
            pltpu.make_async_copy(src_hbm.at[pl.ds(0, _SG)],
                                  src_i.at[pl.ds(0, _SG)], si).wait()
            pltpu.make_async_copy(dst_hbm.at[pl.ds(0, _SG)],
                                  dst_i.at[pl.ds(0, _SG)], si).wait()

        pltpu.sync_copy(src_hbm.at[pl.ds(row0, _SG)],
                        src_i.at[pl.ds(0, _SG)])
        pltpu.sync_copy(dst_hbm.at[pl.ds(row0, _SG)],
                        dst_i.at[pl.ds(0, _SG)])
        _fill_zeros_2d(zbuf, 8, d)
        r0 = s * rows_per_sub

        def zloop(i, carry):
            pltpu.sync_copy(zbuf, acc_sh.at[pl.ds(r0 + i * 8, 8)])
            return carry

        lax.fori_loop(0, rows_per_sub // 8, zloop, 0)
        plsc.subcore_barrier()

        for k in range(_SG):
            b = k % 2
            if k >= 2:
                s_wait(b)
            g_start(k, b)
            if k >= 1:
                g_wait(b ^ 1)
                s_start(k - 1, b ^ 1)
        idx_prefetch(1, _SG)

        def body(sg, carry):
            idx_wait()
            hb = lax.rem(sg, 2) * _SG
            ob = _SG - hb
            for k in range(_SG):
                b = k % 2
                s_wait(b)
                g_start(hb + k, b)
                g_wait(b ^ 1)
                if k == 0:
                    s_start(ob + _SG - 1, b ^ 1)
                else:
                    s_start(hb + k - 1, b ^ 1)
            idx_prefetch(sg + 1, ob)
            return carry

        lax.fori_loop(1, nsg, body, 0)

        hb_last = ((nsg - 1) % 2) * _SG
        g_wait(1)
        s_start(hb_last + _SG - 1, 1)
        s_wait(0)
        s_wait(1)
        idx_wait()

        plsc.subcore_barrier()
        pltpu.sync_copy(acc_sh.at[pl.ds(r0, rows_per_sub)],
                        out_hbm.at[c, pl.ds(r0, rows_per_sub)])

    return agg_kernel(hs, src2, dst2)


def _tc_prep(x_pad, W1, degp, *, n_pad, f_in, h):
    def body(x_ref, w_ref, degp_ref, hs_ref, dinv_ref):
        deg = degp_ref[0, :] + degp_ref[1, :]
        dinv = lax.rsqrt(deg + 1.0)
        hw = jnp.dot(x_ref[...], w_ref[...], preferred_element_type=jnp.float32)
        hs_ref[...] = hw * dinv[:, None]
        dinv_ref[...] = dinv

    return pl.pallas_call(
        body,
        grid=(n_pad // _BR,),
        in_specs=[
            pl.BlockSpec((_BR, f_in), lambda i: (i, 0)),
            pl.BlockSpec((f_in, h), lambda i: (0, 0)),
            pl.BlockSpec((_NSC, _BR), lambda i: (0, i)),
        ],
        out_specs=[
            pl.BlockSpec((_BR, h), lambda i: (i, 0)),
            pl.BlockSpec((_BR,), lambda i: (i,)),
        ],
        out_shape=[
            jax.ShapeDtypeStruct((n_pad, h), jnp.float32),
            jax.ShapeDtypeStruct((n_pad,), jnp.float32),
        ],
    )(x_pad, W1, degp)


def _tc_mid(p1, hs1, dinv, b1, W2, *, n_pad, h):
    def body(p_ref, hs1_ref, dinv_ref, b1_ref, w2_ref, hs2_ref):
        dinv = dinv_ref[...]
        agg = p_ref[0] + p_ref[1] + hs1_ref[...]
        h1 = jnp.maximum(agg * dinv[:, None] + b1_ref[...][None, :], 0.0)
        hs2_ref[...] = (
            jnp.dot(h1, w2_ref[...], preferred_element_type=jnp.float32)
            * dinv[:, None])

    return pl.pallas_call(
        body,
        grid=(n_pad // _BR,),
        in_specs=[
            pl.BlockSpec((_NSC, _BR, h), lambda i: (0, i, 0)),
            pl.BlockSpec((_BR, h), lambda i: (i, 0)),
            pl.BlockSpec((_BR,), lambda i: (i,)),
            pl.BlockSpec((h,), lambda i: (0,)),
            pl.BlockSpec((h, h), lambda i: (0, 0)),
        ],
        out_specs=pl.BlockSpec((_BR, h), lambda i: (i, 0)),
        out_shape=jax.ShapeDtypeStruct((n_pad, h), jnp.float32),
    )(p1, hs1, dinv, b1, W2)


def _tc_final(p2, hs2, dinv, b2, batch_pad, gf, Wl1a, Wl1b, bl1, Wl2, bl2,
              *, n_pad, h, ng, g, nc):
    nblocks = n_pad // _BR

    def body(p_ref, hs2_ref, dinv_ref, b2_ref, batch_ref, gf_ref,
             wl1a_ref, wl1b_ref, bl1_ref, wl2_ref, bl2_ref,
             z_ref, pooled_acc, counts_acc):
        i = pl.program_id(0)

        @pl.when(i == 0)
        def _():
            pooled_acc[...] = jnp.zeros_like(pooled_acc)
            counts_acc[...] = jnp.zeros_like(counts_acc)

        dinv = dinv_ref[...]
        h2 = ((p_ref[0] + p_ref[1] + hs2_ref[...]) * dinv[:, None]
              + b2_ref[...][None, :])
        b = batch_ref[...]
        gids = lax.broadcasted_iota(jnp.int32, (ng, _BR), 0)
        onehot = (b[None, :] == gids).astype(jnp.float32)
        pooled_acc[...] += jnp.dot(onehot, h2,
                                   preferred_element_type=jnp.float32)
        counts_acc[...] += jnp.sum(onehot, axis=1)[:, None]

        @pl.when(i == nblocks - 1)
        def _():
            pooled = pooled_acc[...] / jnp.maximum(counts_acc[...], 1.0)
            t = (jnp.dot(pooled, wl1a_ref[...],
                         preferred_element_type=jnp.float32)
                 + jnp.dot(gf_ref[...], wl1b_ref[...],
                           preferred_element_type=jnp.float32)
                 + bl1_ref[...][None, :])
            t = jnp.maximum(t, 0.0)
            z_ref[...] = (jnp.dot(t, wl2_ref[...],
                                  preferred_element_type=jnp.float32)
                          + bl2_ref[...][None, :])

    return pl.pallas_call(
        body,
        grid=(nblocks,),
        in_specs=[
            pl.BlockSpec((_NSC, _BR, h), lambda i: (0, i, 0)),
            pl.BlockSpec((_BR, h), lambda i: (i, 0)),
            pl.BlockSpec((_BR,), lambda i: (i,)),
            pl.BlockSpec((h,), lambda i: (0,)),
            pl.BlockSpec((_BR,), lambda i: (i,)),
            pl.BlockSpec((ng, g), lambda i: (0, 0)),
            pl.BlockSpec((h, h), lambda i: (0, 0)),
            pl.BlockSpec((g, h), lambda i: (0, 0)),
            pl.BlockSpec((h,), lambda i: (0,)),
            pl.BlockSpec((h, nc), lambda i: (0, 0)),
            pl.BlockSpec((nc,), lambda i: (0,)),
        ],
        out_specs=pl.BlockSpec((ng, nc), lambda i: (0, 0)),
        out_shape=jax.ShapeDtypeStruct((ng, nc), jnp.float32),
        scratch_shapes=[
            pltpu.VMEM((ng, h), jnp.float32),
            pltpu.VMEM((ng, h), jnp.float32),
        ],
    )(p2, hs2, dinv, b2, batch_pad, gf, Wl1a, Wl1b, bl1, Wl2, bl2)


def kernel(x, edge_index, batch, global_features, W1, b1, W2, b2,
           Wl1, bl1, Wl2, bl2):
    n, f_in = x.shape
    h = W1.shape[1]
    e = edge_index.shape[1]
    ng, g = global_features.shape
    nc = Wl2.shape[1]

    n_pad = -(-(n + 1) // _BR) * _BR
    e_chunk = _NW * _CH * _SG
    e_pad = -(-e // e_chunk) * e_chunk

    n_dump = n_pad - n
    pad_idx = n + jnp.arange(e_pad - e, dtype=jnp.int32) % n_dump
    src = jnp.concatenate([edge_index[0], pad_idx])
    dst = jnp.concatenate([edge_index[1], pad_idx])

    x_pad = jnp.pad(x, ((0, n_pad - n), (0, 0)))
    batch_pad = jnp.pad(batch, (0, n_pad - n), constant_values=-1)

    src2 = src.reshape(e_pad // _CH, _CH)
    dst2 = dst.reshape(e_pad // _CH, _CH)

    degp = _sc_degree(dst2, n_pad=n_pad, e_pad=e_pad)
    hs1, dinv = _tc_prep(x_pad, W1, degp, n_pad=n_pad, f_in=f_in, h=h)
    p1 = _sc_aggregate(hs1, src2, dst2, n_pad=n_pad, e_pad=e_pad, d=h)
    hs2 = _tc_mid(p1, hs1, dinv, b1, W2, n_pad=n_pad, h=h)
    p2 = _sc_aggregate(hs2, src2, dst2, n_pad=n_pad, e_pad=e_pad, d=h)
    z = _tc_final(p2, hs2, dinv, b2, batch_pad, global_features,
                  Wl1[:h], Wl1[h:], bl1, Wl2, bl2,
                  n_pad=n_pad, h=h, ng=ng, g=g, nc=nc)
    return z


# speedup vs baseline: 34.9687x; 1.0320x over previous
import functools

import jax
import jax.numpy as jnp
from jax import lax
from jax.experimental import pallas as pl
from jax.experimental.pallas import tpu as pltpu
from jax.experimental.pallas import tpu_sc as plsc

_NSC = 2
_NSUB = 16
_NW = _NSC * _NSUB
_CH = 128
_BR = 1024


def _sc_mesh():
    return plsc.VectorSubcoreMesh(core_axis_name="c", subcore_axis_name="s")


def _fill_zeros_1d(ref, cols):
    z16 = jnp.zeros((16,), jnp.float32)
    for k in range(cols // 16):
        ref[pl.ds(k * 16, 16)] = z16


def _fill_zeros_2d(ref, rows, cols):
    z16 = jnp.zeros((16,), jnp.float32)
    for r in range(rows):
        for k in range(cols // 16):
            ref[r, pl.ds(k * 16, 16)] = z16


@functools.partial(jax.jit, static_argnames=("n_pad", "e_pad"))
def _sc_degree(dst2, *, n_pad, e_pad):
    rows_per_sub = n_pad // _NSUB
    e_per_w = e_pad // _NW
    nch = e_per_w // _CH
    nsg = nch // _SG
    assert nch == nsg * _SG and nsg >= 2
    total_rows = e_pad // _CH

    @functools.partial(
        pl.kernel,
        mesh=_sc_mesh(),
        out_type=jax.ShapeDtypeStruct((_NSC, n_pad), jnp.float32),
        scratch_types=[
            pltpu.VMEM((2 * _SG, _CH), jnp.int32),
            pltpu.VMEM((_CH,), jnp.float32),
            pltpu.VMEM((rows_per_sub,), jnp.float32),
            pltpu.VMEM_SHARED((n_pad,), jnp.float32),
            pltpu.SemaphoreType.DMA,
            pltpu.SemaphoreType.DMA,
        ],
    )
    def deg_kernel(dst_hbm, out_hbm, dst_i, ones_v, zb, deg_sh, ss, si):
        c = lax.axis_index("c")
        s = lax.axis_index("s")
        wid = c * _NSUB + s
        row0 = wid * nch
        one16 = jnp.ones((16,), jnp.float32)
        for k in range(_CH // 16):
            ones_v[pl.ds(k * 16, 16)] = one16

        def s_start(row):
            pltpu.async_copy(ones_v, deg_sh.at[dst_i.at[row]], ss, add=True)

        def s_drain():
            for _ in range(_SG):
                pltpu.make_async_copy(ones_v, deg_sh.at[pl.ds(0, _CH)],
                                      ss).wait()

        def idx_prefetch(sg_next, half_base):
            rstart = jnp.minimum(row0 + sg_next * _SG,
                                 jnp.int32(total_rows - _SG))
            pltpu.async_copy(dst_hbm.at[pl.ds(rstart, _SG)],
                             dst_i.at[pl.ds(half_base, _SG)], si)

        def idx_wait():
            pltpu.make_async_copy(dst_hbm.at[pl.ds(0, _SG)],
                                  dst_i.at[pl.ds(0, _SG)], si).wait()

        pltpu.sync_copy(dst_hbm.at[pl.ds(row0, _SG)],
                        dst_i.at[pl.ds(0, _SG)])
        _fill_zeros_1d(zb, rows_per_sub)
        r0 = s * rows_per_sub
        pltpu.sync_copy(zb, deg_sh.at[pl.ds(r0, rows_per_sub)])
        plsc.subcore_barrier()

        for k in range(_SG):
            s_start(k)
        idx_prefetch(1, _SG)

        def body(sg, carry):
            idx_wait()
            hb = lax.rem(sg, 2) * _SG
            s_drain()
            for k in range(_SG):
                s_start(hb + k)
            idx_prefetch(sg + 1, _SG - hb)
            return carry

        lax.fori_loop(1, nsg, body, 0)
        s_drain()
        idx_wait()

        plsc.subcore_barrier()
        pltpu.sync_copy(deg_sh.at[pl.ds(r0, rows_per_sub)],
                        out_hbm.at[c, pl.ds(r0, rows_per_sub)])

    return deg_kernel(dst2)


_SG = 16


@functools.partial(jax.jit, static_argnames=("n_pad", "e_pad", "d"))
def _sc_aggregate(hs, src2, dst2, *, n_pad, e_pad, d):
    rows_per_sub = n_pad // _NSUB
    e_per_w = e_pad // _NW
    nch = e_per_w // _CH
    nsg = nch // _SG
    assert nch == nsg * _SG and nsg >= 2
    total_rows = e_pad // _CH

    @functools.partial(
        pl.kernel,
        mesh=_sc_mesh(),
        out_type=jax.ShapeDtypeStruct((_NSC, n_pad, d), jnp.float32),
        scratch_types=[
            pltpu.VMEM((2 * _SG, _CH), jnp.int32),
            pltpu.VMEM((2 * _SG, _CH), jnp.int32),
            pltpu.VMEM((_CH, d), jnp.float32),
            pltpu.VMEM((_CH, d), jnp.float32),
            pltpu.VMEM((64, d), jnp.float32),
            pltpu.VMEM_SHARED((n_pad, d), jnp.float32),
            pltpu.SemaphoreType.DMA,
            pltpu.SemaphoreType.DMA,
            pltpu.SemaphoreType.DMA,
            pltpu.SemaphoreType.DMA,
            pltpu.SemaphoreType.DMA,
        ],
    )
    def agg_kernel(hs_hbm, src_hbm, dst_hbm, out_hbm,
                   src_i, dst_i, d0, d1, zbuf, acc_sh,
                   sg0, sg1, ss0, ss1, si):
        data = (d0, d1)
        gsem = (sg0, sg1)
        ssem = (ss0, ss1)
        c = lax.axis_index("c")
        s = lax.axis_index("s")
        wid = c * _NSUB + s
        row0 = wid * nch

        def g_start(row, b):
            pltpu.async_copy(hs_hbm.at[src_i.at[row]], data[b], gsem[b])

        def g_wait(b):
            pltpu.make_async_copy(hs_hbm.at[pl.ds(0, _CH)], data[b],
                                  gsem[b]).wait()

        def s_start(row, b):
            pltpu.async_copy(data[b], acc_sh.at[dst_i.at[row]], ssem[b],
                             add=True)

        def s_wait(b):
            pltpu.make_async_copy(data[b], acc_sh.at[pl.ds(0, _CH)],
                                  ssem[b]).wait()

        def idx_prefetch(sg_next, half_base):
            rstart = jnp.minimum(row0 + sg_next * _SG,
                                 jnp.int32(total_rows - _SG))
            pltpu.async_copy(src_hbm.at[pl.ds(rstart, _SG)],
                             src_i.at[pl.ds(half_base, _SG)], si)
            pltpu.async_copy(dst_hbm.at[pl.ds(rstart, _SG)],
                             dst_i.at[pl.ds(half_base, _SG)], si)

        def idx_wait():
            pltpu.make_async_copy(src_hbm.at[pl.ds(0, _SG)],
                                  src_i.at[pl.ds(0, _SG)], si).wait()
            pltpu.make_async_copy(dst_hbm.at[pl.ds(0, _SG)],
                                  dst_i.at[pl.ds(0, _SG)], si).wait()

        pltpu.sync_copy(src_hbm.at[pl.ds(row0, _SG)],
                        src_i.at[pl.ds(0, _SG)])
        pltpu.sync_copy(dst_hbm.at[pl.ds(row0, _SG)],
                        dst_i.at[pl.ds(0, _SG)])
        _fill_zeros_2d(zbuf, 64, d)
        r0 = s * rows_per_sub
        nz = rows_per_sub // 64
        for i in range(nz):
            pltpu.async_copy(zbuf, acc_sh.at[pl.ds(r0 + i * 64, 64)], ss0)
        for i in range(nz):
            pltpu.make_async_copy(zbuf, acc_sh.at[pl.ds(r0, 64)], ss0).wait()
        plsc.subcore_barrier()

        for k in range(_SG):
            b = k % 2
            if k >= 2:
                s_wait(b)
            g_start(k, b)
            if k >= 1:
                g_wait(b ^ 1)
                s_start(k - 1, b ^ 1)
        idx_prefetch(1, _SG)

        def body(sg, carry):
            idx_wait()
            hb = lax.rem(sg, 2) * _SG
            ob = _SG - hb
            for k in range(_SG):
                b = k % 2
                s_wait(b)
                g_start(hb + k, b)
                g_wait(b ^ 1)
                if k == 0:
                    s_start(ob + _SG - 1, b ^ 1)
                else:
                    s_start(hb + k - 1, b ^ 1)
            idx_prefetch(sg + 1, ob)
            return carry

        lax.fori_loop(1, nsg, body, 0)

        hb_last = ((nsg - 1) % 2) * _SG
        g_wait(1)
        s_start(hb_last + _SG - 1, 1)
        s_wait(0)
        s_wait(1)
        idx_wait()

        plsc.subcore_barrier()
        pltpu.sync_copy(acc_sh.at[pl.ds(r0, rows_per_sub)],
                        out_hbm.at[c, pl.ds(r0, rows_per_sub)])

    return agg_kernel(hs, src2, dst2)


def _tc_prep(x_pad, W1, degp, *, n_pad, f_in, h):
    def body(x_ref, w_ref, degp_ref, hs_ref, dinv_ref):
        deg = degp_ref[0, :] + degp_ref[1, :]
        dinv = lax.rsqrt(deg + 1.0)
        hw = jnp.dot(x_ref[...], w_ref[...], preferred_element_type=jnp.float32)
        hs_ref[...] = hw * dinv[:, None]
        dinv_ref[...] = dinv

    return pl.pallas_call(
        body,
        grid=(n_pad // _BR,),
        in_specs=[
            pl.BlockSpec((_BR, f_in), lambda i: (i, 0)),
            pl.BlockSpec((f_in, h), lambda i: (0, 0)),
            pl.BlockSpec((_NSC, _BR), lambda i: (0, i)),
        ],
        out_specs=[
            pl.BlockSpec((_BR, h), lambda i: (i, 0)),
            pl.BlockSpec((_BR,), lambda i: (i,)),
        ],
        out_shape=[
            jax.ShapeDtypeStruct((n_pad, h), jnp.float32),
            jax.ShapeDtypeStruct((n_pad,), jnp.float32),
        ],
    )(x_pad, W1, degp)


def _tc_mid(p1, hs1, dinv, b1, W2, *, n_pad, h):
    def body(p_ref, hs1_ref, dinv_ref, b1_ref, w2_ref, hs2_ref):
        dinv = dinv_ref[...]
        agg = p_ref[0] + p_ref[1] + hs1_ref[...]
        h1 = jnp.maximum(agg * dinv[:, None] + b1_ref[...][None, :], 0.0)
        hs2_ref[...] = (
            jnp.dot(h1, w2_ref[...], preferred_element_type=jnp.float32)
            * dinv[:, None])

    return pl.pallas_call(
        body,
        grid=(n_pad // _BR,),
        in_specs=[
            pl.BlockSpec((_NSC, _BR, h), lambda i: (0, i, 0)),
            pl.BlockSpec((_BR, h), lambda i: (i, 0)),
            pl.BlockSpec((_BR,), lambda i: (i,)),
            pl.BlockSpec((h,), lambda i: (0,)),
            pl.BlockSpec((h, h), lambda i: (0, 0)),
        ],
        out_specs=pl.BlockSpec((_BR, h), lambda i: (i, 0)),
        out_shape=jax.ShapeDtypeStruct((n_pad, h), jnp.float32),
    )(p1, hs1, dinv, b1, W2)


def _tc_final(p2, hs2, dinv, b2, batch_pad, gf, Wl1a, Wl1b, bl1, Wl2, bl2,
              *, n_pad, h, ng, g, nc):
    nblocks = n_pad // _BR

    def body(p_ref, hs2_ref, dinv_ref, b2_ref, batch_ref, gf_ref,
             wl1a_ref, wl1b_ref, bl1_ref, wl2_ref, bl2_ref,
             z_ref, pooled_acc, counts_acc):
        i = pl.program_id(0)

        @pl.when(i == 0)
        def _():
            pooled_acc[...] = jnp.zeros_like(pooled_acc)
            counts_acc[...] = jnp.zeros_like(counts_acc)

        dinv = dinv_ref[...]
        h2 = ((p_ref[0] + p_ref[1] + hs2_ref[...]) * dinv[:, None]
              + b2_ref[...][None, :])
        b = batch_ref[...]
        gids = lax.broadcasted_iota(jnp.int32, (ng, _BR), 0)
        onehot = (b[None, :] == gids).astype(jnp.float32)
        pooled_acc[...] += jnp.dot(onehot, h2,
                                   preferred_element_type=jnp.float32)
        counts_acc[...] += jnp.sum(onehot, axis=1)[:, None]

        @pl.when(i == nblocks - 1)
        def _():
            pooled = pooled_acc[...] / jnp.maximum(counts_acc[...], 1.0)
            t = (jnp.dot(pooled, wl1a_ref[...],
                         preferred_element_type=jnp.float32)
                 + jnp.dot(gf_ref[...], wl1b_ref[...],
                           preferred_element_type=jnp.float32)
                 + bl1_ref[...][None, :])
            t = jnp.maximum(t, 0.0)
            z_ref[...] = (jnp.dot(t, wl2_ref[...],
                                  preferred_element_type=jnp.float32)
                          + bl2_ref[...][None, :])

    return pl.pallas_call(
        body,
        grid=(nblocks,),
        in_specs=[
            pl.BlockSpec((_NSC, _BR, h), lambda i: (0, i, 0)),
            pl.BlockSpec((_BR, h), lambda i: (i, 0)),
            pl.BlockSpec((_BR,), lambda i: (i,)),
            pl.BlockSpec((h,), lambda i: (0,)),
            pl.BlockSpec((_BR,), lambda i: (i,)),
            pl.BlockSpec((ng, g), lambda i: (0, 0)),
            pl.BlockSpec((h, h), lambda i: (0, 0)),
            pl.BlockSpec((g, h), lambda i: (0, 0)),
            pl.BlockSpec((h,), lambda i: (0,)),
            pl.BlockSpec((h, nc), lambda i: (0, 0)),
            pl.BlockSpec((nc,), lambda i: (0,)),
        ],
        out_specs=pl.BlockSpec((ng, nc), lambda i: (0, 0)),
        out_shape=jax.ShapeDtypeStruct((ng, nc), jnp.float32),
        scratch_shapes=[
            pltpu.VMEM((ng, h), jnp.float32),
            pltpu.VMEM((ng, h), jnp.float32),
        ],
    )(p2, hs2, dinv, b2, batch_pad, gf, Wl1a, Wl1b, bl1, Wl2, bl2)


def kernel(x, edge_index, batch, global_features, W1, b1, W2, b2,
           Wl1, bl1, Wl2, bl2):
    n, f_in = x.shape
    h = W1.shape[1]
    e = edge_index.shape[1]
    ng, g = global_features.shape
    nc = Wl2.shape[1]

    n_pad = -(-(n + 1) // _BR) * _BR
    e_chunk = _NW * _CH * _SG
    e_pad = -(-e // e_chunk) * e_chunk

    n_dump = n_pad - n
    pad_idx = n + jnp.arange(e_pad - e, dtype=jnp.int32) % n_dump
    src = jnp.concatenate([edge_index[0], pad_idx])
    dst = jnp.concatenate([edge_index[1], pad_idx])

    x_pad = jnp.pad(x, ((0, n_pad - n), (0, 0)))
    batch_pad = jnp.pad(batch, (0, n_pad - n), constant_values=-1)

    src2 = src.reshape(e_pad // _CH, _CH)
    dst2 = dst.reshape(e_pad // _CH, _CH)

    degp = _sc_degree(dst2, n_pad=n_pad, e_pad=e_pad)
    hs1, dinv = _tc_prep(x_pad, W1, degp, n_pad=n_pad, f_in=f_in, h=h)
    p1 = _sc_aggregate(hs1, src2, dst2, n_pad=n_pad, e_pad=e_pad, d=h)
    hs2 = _tc_mid(p1, hs1, dinv, b1, W2, n_pad=n_pad, h=h)
    p2 = _sc_aggregate(hs2, src2, dst2, n_pad=n_pad, e_pad=e_pad, d=h)
    z = _tc_final(p2, hs2, dinv, b2, batch_pad, global_features,
                  Wl1[:h], Wl1[h:], bl1, Wl2, bl2,
                  n_pad=n_pad, h=h, ng=ng, g=g, nc=nc)
    return z


# speedup vs baseline: 35.2443x; 1.0079x over previous
import functools

import jax
import jax.numpy as jnp
from jax import lax
from jax.experimental import pallas as pl
from jax.experimental.pallas import tpu as pltpu
from jax.experimental.pallas import tpu_sc as plsc

_NSC = 2
_NSUB = 16
_NW = _NSC * _NSUB
_CH = 128
_BR = 1024


def _sc_mesh():
    return plsc.VectorSubcoreMesh(core_axis_name="c", subcore_axis_name="s")


def _fill_zeros_1d(ref, cols):
    z16 = jnp.zeros((16,), jnp.float32)
    for k in range(cols // 16):
        ref[pl.ds(k * 16, 16)] = z16


def _fill_zeros_2d(ref, rows, cols):
    z16 = jnp.zeros((16,), jnp.float32)
    for r in range(rows):
        for k in range(cols // 16):
            ref[r, pl.ds(k * 16, 16)] = z16


@functools.partial(jax.jit, static_argnames=("n_pad", "e_pad"))
def _sc_degree(dst2, *, n_pad, e_pad):
    rows_per_sub = n_pad // _NSUB
    e_per_w = e_pad // _NW
    nch = e_per_w // _CH
    nsg = nch // _SG
    assert nch == nsg * _SG and nsg >= 2
    total_rows = e_pad // _CH

    @functools.partial(
        pl.kernel,
        mesh=_sc_mesh(),
        out_type=jax.ShapeDtypeStruct((_NSC, n_pad), jnp.float32),
        scratch_types=[
            pltpu.VMEM((2 * _SG, _CH), jnp.int32),
            pltpu.VMEM((_CH,), jnp.float32),
            pltpu.VMEM((rows_per_sub,), jnp.float32),
            pltpu.VMEM_SHARED((n_pad,), jnp.float32),
            pltpu.SemaphoreType.DMA,
            pltpu.SemaphoreType.DMA,
        ],
    )
    def deg_kernel(dst_hbm, out_hbm, dst_i, ones_v, zb, deg_sh, ss, si):
        c = lax.axis_index("c")
        s = lax.axis_index("s")
        wid = c * _NSUB + s
        row0 = wid * nch
        one16 = jnp.ones((16,), jnp.float32)
        for k in range(_CH // 16):
            ones_v[pl.ds(k * 16, 16)] = one16

        def s_start(row):
            pltpu.async_copy(ones_v, deg_sh.at[dst_i.at[row]], ss, add=True)

        def s_drain():
            for _ in range(_SG):
                pltpu.make_async_copy(ones_v, deg_sh.at[pl.ds(0, _CH)],
                                      ss).wait()

        def idx_prefetch(sg_next, half_base):
            rstart = jnp.minimum(row0 + sg_next * _SG,
                                 jnp.int32(total_rows - _SG))
            pltpu.async_copy(dst_hbm.at[pl.ds(rstart, _SG)],
                             dst_i.at[pl.ds(half_base, _SG)], si)

        def idx_wait():
            pltpu.make_async_copy(dst_hbm.at[pl.ds(0, _SG)],
                                  dst_i.at[pl.ds(0, _SG)], si).wait()

        pltpu.sync_copy(dst_hbm.at[pl.ds(row0, _SG)],
                        dst_i.at[pl.ds(0, _SG)])
        _fill_zeros_1d(zb, rows_per_sub)
        r0 = s * rows_per_sub
        pltpu.sync_copy(zb, deg_sh.at[pl.ds(r0, rows_per_sub)])
        plsc.subcore_barrier()

        for k in range(_SG):
            s_start(k)
        idx_prefetch(1, _SG)

        def body(sg, carry):
            idx_wait()
            hb = lax.rem(sg, 2) * _SG
            s_drain()
            for k in range(_SG):
                s_start(hb + k)
            idx_prefetch(sg + 1, _SG - hb)
            return carry

        lax.fori_loop(1, nsg, body, 0)
        s_drain()
        idx_wait()

        plsc.subcore_barrier()
        pltpu.sync_copy(deg_sh.at[pl.ds(r0, rows_per_sub)],
                        out_hbm.at[c, pl.ds(r0, rows_per_sub)])

    return deg_kernel(dst2)


_SG = 16


@functools.partial(jax.jit, static_argnames=("n_pad", "e_pad", "d"))
def _sc_aggregate(hs, src2, dst2, *, n_pad, e_pad, d):
    rows_per_sub = n_pad // _NSUB
    e_per_w = e_pad // _NW
    nch = e_per_w // _CH
    nsg = nch // _SG
    assert nch == nsg * _SG and nsg >= 2
    total_rows = e_pad // _CH

    @functools.partial(
        pl.kernel,
        mesh=_sc_mesh(),
        out_type=jax.ShapeDtypeStruct((_NSC, n_pad, d), jnp.float32),
        scratch_types=[
            pltpu.VMEM((2 * _SG, _CH), jnp.int32),
            pltpu.VMEM((2 * _SG, _CH), jnp.int32),
            pltpu.VMEM((_CH, d), jnp.float32),
            pltpu.VMEM((_CH, d), jnp.float32),
            pltpu.VMEM((64, d), jnp.float32),
            pltpu.VMEM_SHARED((n_pad, d), jnp.float32),
            pltpu.SemaphoreType.DMA,
            pltpu.SemaphoreType.DMA,
            pltpu.SemaphoreType.DMA,
            pltpu.SemaphoreType.DMA,
            pltpu.SemaphoreType.DMA,
        ],
    )
    def agg_kernel(hs_hbm, src_hbm, dst_hbm, out_hbm,
                   src_i, dst_i, d0, d1, zbuf, acc_sh,
                   sg0, sg1, ss0, ss1, si):
        data = (d0, d1)
        gsem = (sg0, sg1)
        ssem = (ss0, ss1)
        c = lax.axis_index("c")
        s = lax.axis_index("s")
        wid = c * _NSUB + s
        row0 = wid * nch

        def g_start(row, b):
            pltpu.async_copy(hs_hbm.at[src_i.at[row]], data[b], gsem[b])

        def g_wait(b):
            pltpu.make_async_copy(hs_hbm.at[pl.ds(0, _CH)], data[b],
                                  gsem[b]).wait()

        def s_start(row, b):
            pltpu.async_copy(data[b], acc_sh.at[dst_i.at[row]], ssem[b],
                             add=True)

        def s_wait(b):
            pltpu.make_async_copy(data[b], acc_sh.at[pl.ds(0, _CH)],
                                  ssem[b]).wait()

        def idx_prefetch(sg_next, half_base):
            rstart = jnp.minimum(row0 + sg_next * _SG,
                                 jnp.int32(total_rows - _SG))
            pltpu.async_copy(src_hbm.at[pl.ds(rstart, _SG)],
                             src_i.at[pl.ds(half_base, _SG)], si)
            pltpu.async_copy(dst_hbm.at[pl.ds(rstart, _SG)],
                             dst_i.at[pl.ds(half_base, _SG)], si)

        def idx_wait():
            pltpu.make_async_copy(src_hbm.at[pl.ds(0, _SG)],
                                  src_i.at[pl.ds(0, _SG)], si).wait()
            pltpu.make_async_copy(dst_hbm.at[pl.ds(0, _SG)],
                                  dst_i.at[pl.ds(0, _SG)], si).wait()

        pltpu.sync_copy(src_hbm.at[pl.ds(row0, _SG)],
                        src_i.at[pl.ds(0, _SG)])
        pltpu.sync_copy(dst_hbm.at[pl.ds(row0, _SG)],
                        dst_i.at[pl.ds(0, _SG)])
        _fill_zeros_2d(zbuf, 64, d)
        r0 = s * rows_per_sub
        nz = rows_per_sub // 64
        for i in range(nz):
            pltpu.async_copy(zbuf, acc_sh.at[pl.ds(r0 + i * 64, 64)], ss0)
        for i in range(nz):
            pltpu.make_async_copy(zbuf, acc_sh.at[pl.ds(r0, 64)], ss0).wait()
        plsc.subcore_barrier()

        for k in range(_SG):
            b = k % 2
            if k >= 2:
                s_wait(b)
            g_start(k, b)
            if k >= 1:
                g_wait(b ^ 1)
                s_start(k - 1, b ^ 1)
        idx_prefetch(1, _SG)

        def body(sg, carry):
            idx_wait()
            hb = lax.rem(sg, 2) * _SG
            ob = _SG - hb
            for k in range(_SG):
                b = k % 2
                s_wait(b)
                g_start(hb + k, b)
                g_wait(b ^ 1)
                if k == 0:
                    s_start(ob + _SG - 1, b ^ 1)
                else:
                    s_start(hb + k - 1, b ^ 1)
            idx_prefetch(sg + 1, ob)
            return carry

        lax.fori_loop(1, nsg, body, 0)

        hb_last = ((nsg - 1) % 2) * _SG
        g_wait(1)
        s_start(hb_last + _SG - 1, 1)
        s_wait(0)
        s_wait(1)
        idx_wait()

        plsc.subcore_barrier()
        pltpu.sync_copy(acc_sh.at[pl.ds(r0, rows_per_sub)],
                        out_hbm.at[c, pl.ds(r0, rows_per_sub)])

    return agg_kernel(hs, src2, dst2)


def _tc_prep(x_pad, W1, degp, *, n_pad, f_in, h):
    def body(x_ref, w_ref, degp_ref, hs_ref, dinv_ref):
        deg = degp_ref[0, :] + degp_ref[1, :]
        dinv = lax.rsqrt(deg + 1.0)
        hw = jnp.dot(x_ref[...], w_ref[...], preferred_element_type=jnp.float32)
        hs_ref[...] = hw * dinv[:, None]
        dinv_ref[...] = dinv

    return pl.pallas_call(
        body,
        grid=(n_pad // _BR,),
        in_specs=[
            pl.BlockSpec((_BR, f_in), lambda i: (i, 0)),
            pl.BlockSpec((f_in, h), lambda i: (0, 0)),
            pl.BlockSpec((_NSC, _BR), lambda i: (0, i)),
        ],
        out_specs=[
            pl.BlockSpec((_BR, h), lambda i: (i, 0)),
            pl.BlockSpec((_BR,), lambda i: (i,)),
        ],
        out_shape=[
            jax.ShapeDtypeStruct((n_pad, h), jnp.float32),
            jax.ShapeDtypeStruct((n_pad,), jnp.float32),
        ],
    )(x_pad, W1, degp)


def _tc_mid(p1, hs1, dinv, b1, W2, *, n_pad, h):
    def body(p_ref, hs1_ref, dinv_ref, b1_ref, w2_ref, hs2_ref):
        dinv = dinv_ref[...]
        agg = p_ref[0] + p_ref[1] + hs1_ref[...]
        h1 = jnp.maximum(agg * dinv[:, None] + b1_ref[...][None, :], 0.0)
        hs2_ref[...] = (
            jnp.dot(h1, w2_ref[...], preferred_element_type=jnp.float32)
            * dinv[:, None])

    return pl.pallas_call(
        body,
        grid=(n_pad // _BR,),
        in_specs=[
            pl.BlockSpec((_NSC, _BR, h), lambda i: (0, i, 0)),
            pl.BlockSpec((_BR, h), lambda i: (i, 0)),
            pl.BlockSpec((_BR,), lambda i: (i,)),
            pl.BlockSpec((h,), lambda i: (0,)),
            pl.BlockSpec((h, h), lambda i: (0, 0)),
        ],
        out_specs=pl.BlockSpec((_BR, h), lambda i: (i, 0)),
        out_shape=jax.ShapeDtypeStruct((n_pad, h), jnp.float32),
    )(p1, hs1, dinv, b1, W2)


def _tc_final(p2, hs2, dinv, b2, batch, gf, Wl1a, Wl1b, bl1, Wl2, bl2,
              *, n, n_pad, h, ng, g, nc):
    nblocks = n_pad // _BR

    def body(p_ref, hs2_ref, dinv_ref, b2_ref, batch_ref, gf_ref,
             wl1a_ref, wl1b_ref, bl1_ref, wl2_ref, bl2_ref,
             z_ref, pooled_acc, counts_acc):
        i = pl.program_id(0)

        @pl.when(i == 0)
        def _():
            pooled_acc[...] = jnp.zeros_like(pooled_acc)
            counts_acc[...] = jnp.zeros_like(counts_acc)

        dinv = dinv_ref[...]
        rcol = lax.broadcasted_iota(jnp.int32, (_BR, 1), 0) + i * _BR
        h2 = ((p_ref[0] + p_ref[1] + hs2_ref[...]) * dinv[:, None]
              + b2_ref[...][None, :])
        h2 = jnp.where(rcol < n, h2, 0.0)
        b = batch_ref[...]
        gids = lax.broadcasted_iota(jnp.int32, (ng, _BR), 0)
        rrow = lax.broadcasted_iota(jnp.int32, (ng, _BR), 1) + i * _BR
        onehot = ((b[None, :] == gids) & (rrow < n)).astype(jnp.float32)
        pooled_acc[...] += jnp.dot(onehot, h2,
                                   preferred_element_type=jnp.float32)
        counts_acc[...] += jnp.sum(onehot, axis=1)[:, None]

        @pl.when(i == nblocks - 1)
        def _():
            pooled = pooled_acc[...] / jnp.maximum(counts_acc[...], 1.0)
            t = (jnp.dot(pooled, wl1a_ref[...],
                         preferred_element_type=jnp.float32)
                 + jnp.dot(gf_ref[...], wl1b_ref[...],
                           preferred_element_type=jnp.float32)
                 + bl1_ref[...][None, :])
            t = jnp.maximum(t, 0.0)
            z_ref[...] = (jnp.dot(t, wl2_ref[...],
                                  preferred_element_type=jnp.float32)
                          + bl2_ref[...][None, :])

    return pl.pallas_call(
        body,
        grid=(nblocks,),
        in_specs=[
            pl.BlockSpec((_NSC, _BR, h), lambda i: (0, i, 0)),
            pl.BlockSpec((_BR, h), lambda i: (i, 0)),
            pl.BlockSpec((_BR,), lambda i: (i,)),
            pl.BlockSpec((h,), lambda i: (0,)),
            pl.BlockSpec((_BR,), lambda i: (i,)),
            pl.BlockSpec((ng, g), lambda i: (0, 0)),
            pl.BlockSpec((h, h), lambda i: (0, 0)),
            pl.BlockSpec((g, h), lambda i: (0, 0)),
            pl.BlockSpec((h,), lambda i: (0,)),
            pl.BlockSpec((h, nc), lambda i: (0, 0)),
            pl.BlockSpec((nc,), lambda i: (0,)),
        ],
        out_specs=pl.BlockSpec((ng, nc), lambda i: (0, 0)),
        out_shape=jax.ShapeDtypeStruct((ng, nc), jnp.float32),
        scratch_shapes=[
            pltpu.VMEM((ng, h), jnp.float32),
            pltpu.VMEM((ng, h), jnp.float32),
        ],
    )(p2, hs2, dinv, b2, batch, gf, Wl1a, Wl1b, bl1, Wl2, bl2)


def kernel(x, edge_index, batch, global_features, W1, b1, W2, b2,
           Wl1, bl1, Wl2, bl2):
    n, f_in = x.shape
    h = W1.shape[1]
    e = edge_index.shape[1]
    ng, g = global_features.shape
    nc = Wl2.shape[1]

    n_pad = -(-(n + 1) // _BR) * _BR
    e_chunk = _NW * _CH * _SG
    e_pad = -(-e // e_chunk) * e_chunk

    n_dump = n_pad - n
    pad_idx = n + jnp.arange(e_pad - e, dtype=jnp.int32) % n_dump
    src = jnp.concatenate([edge_index[0], pad_idx])
    dst = jnp.concatenate([edge_index[1], pad_idx])

    src2 = src.reshape(e_pad // _CH, _CH)
    dst2 = dst.reshape(e_pad // _CH, _CH)

    degp = _sc_degree(dst2, n_pad=n_pad, e_pad=e_pad)
    hs1, dinv = _tc_prep(x, W1, degp, n_pad=n_pad, f_in=f_in, h=h)
    p1 = _sc_aggregate(hs1, src2, dst2, n_pad=n_pad, e_pad=e_pad, d=h)
    hs2 = _tc_mid(p1, hs1, dinv, b1, W2, n_pad=n_pad, h=h)
    p2 = _sc_aggregate(hs2, src2, dst2, n_pad=n_pad, e_pad=e_pad, d=h)
    z = _tc_final(p2, hs2, dinv, b2, batch, global_features,
                  Wl1[:h], Wl1[h:], bl1, Wl2, bl2,
                  n=n, n_pad=n_pad, h=h, ng=ng, g=g, nc=nc)
    return z
